# baseline (speedup 1.0000x reference)
import functools

import jax
import jax.numpy as jnp
from jax import lax
from jax.experimental import pallas as pl
from jax.experimental.pallas import tpu as pltpu
from jax.experimental.pallas import tpu_sc as plsc

_NC, _NS = 2, 16
_NW = _NC * _NS
_CH = 80
_LANES = 16
_ZB = 8
_R = 4


def _sc_aggregate(x, idx4d, edge_attr):
    n, d = x.shape
    e = edge_attr.shape[0]
    ept = e // _NW
    nchunk = ept // _CH
    rpt = -(-(n // _NS) // _ZB) * _ZB
    owner = n // rpt
    rem = n % rpt
    ngrp = d // _LANES

    mesh = plsc.VectorSubcoreMesh(core_axis_name="c", subcore_axis_name="s")

    @functools.partial(
        pl.kernel,
        out_type=jax.ShapeDtypeStruct((_NC * n, d), jnp.float32),
        mesh=mesh,
        scratch_types=[
            pltpu.VMEM((_R, 2, _CH), jnp.int32),
            pltpu.VMEM((2 * _CH, d), jnp.float32),
            pltpu.VMEM((_CH, d), jnp.float32),
            pltpu.VMEM_SHARED((_NS * rpt, d), jnp.float32),
            pltpu.SemaphoreType.DMA,
            pltpu.SemaphoreType.DMA,
            pltpu.SemaphoreType.DMA,
        ],
    )
    def agg_kernel(x_hbm, idx_hbm, ea_hbm, out_hbm,
                   idx_v, rows_v, ea_v, acc_sh,
                   sem_idx, sem_in, sem_sc):
        c = lax.axis_index("c")
        s = lax.axis_index("s")
        wid = c * _NS + s

        def zrow(r, carry):
            for k in range(ngrp):
                ea_v[r, pl.ds(k * _LANES, _LANES)] = jnp.zeros(
                    (_LANES,), jnp.float32)
            return carry
        lax.fori_loop(0, _CH, zrow, 0)
        row0 = s * rpt
        nfull = rpt // _CH
        for t in range(nfull):
            pltpu.sync_copy(ea_v, acc_sh.at[pl.ds(row0 + t * _CH, _CH), :])
        if rpt % _CH:
            pltpu.sync_copy(
                ea_v.at[pl.ds(0, rpt % _CH), :],
                acc_sh.at[pl.ds(row0 + nfull * _CH, rpt % _CH), :])
        plsc.subcore_barrier()

        base = wid * ept

        def ea_copy(k):
            return pltpu.make_async_copy(
                ea_hbm.at[pl.ds(base + k * _CH, _CH), :], ea_v, sem_in)

        def gather_copy(islot, rslot):
            return pltpu.make_async_copy(
                x_hbm.at[idx_v.at[islot, 0]],
                rows_v.at[pl.ds(rslot * _CH, _CH), :], sem_in)

        def pair_wait(sem):
            pltpu.make_async_copy(
                ea_hbm.at[pl.ds(0, 2 * _CH), :], rows_v, sem).wait()

        def scr_drain():
            pltpu.make_async_copy(
                ea_hbm.at[pl.ds(0, _CH), :],
                rows_v.at[pl.ds(0, _CH), :], sem_sc).wait()

        def idx_copy(k, slot):
            return pltpu.make_async_copy(idx_hbm.at[wid, k], idx_v.at[slot],
                                         sem_idx)

        pltpu.sync_copy(idx_hbm.at[wid, 0], idx_v.at[0])
        gather_copy(0, 0).start()
        ea_copy(0).start()
        idx_copy(1, 1).start()
        idx_copy(2, 2).start()

        def chunk(j, carry):
            b = j % 2
            i = j % _R
            i1 = (j + 1) % _R
            more = j + 1 < nchunk

            @pl.when(j >= 1)
            def _drain_prev():
                scr_drain()
                ea_copy(j).start()

            @pl.when(more)
            def _wait_idx():
                idx_copy(j + 1, i1).wait()

            @pl.when(j + 3 < nchunk)
            def _prefetch_idx():
                idx_copy(j + 3, (j + 3) % _R).start()

            pair_wait(sem_in)

            @pl.when(more)
            def _next_gather():
                gather_copy(i1, 1 - b).start()

            def arow(r, inner):
                row = b * _CH + r
                for g in range(ngrp):
                    sl = pl.ds(g * _LANES, _LANES)
                    rows_v[row, sl] = rows_v[row, sl] + ea_v[r, sl]
                return inner
            lax.fori_loop(0, _CH, arow, 0)

            pltpu.async_copy(rows_v.at[pl.ds(b * _CH, _CH), :],
                             acc_sh.at[idx_v.at[i, 1]], sem_sc, add=True)
            return carry
        lax.fori_loop(0, nchunk, chunk, 0)

        scr_drain()

        plsc.subcore_barrier()

        @pl.when(s < owner)
        def _full():
            pltpu.sync_copy(acc_sh.at[pl.ds(row0, rpt), :],
                            out_hbm.at[pl.ds(c * n + row0, rpt), :])

        if rem:
            @pl.when(s == owner)
            def _ragged():
                pltpu.sync_copy(acc_sh.at[pl.ds(row0, rem), :],
                                out_hbm.at[pl.ds(c * n + row0, rem), :])

    return agg_kernel(x, idx4d, edge_attr)


def _mlp_norm_gelu_residual(parts, x, W1, b1, W2, b2,
                            ln_w, ln_b, gn_w, gn_b, gn_ms, n, d, br):
    nb = n // br
    inv_nd = 1.0 / (n * d)
    inv_n = 1.0 / n

    def body(p0_ref, p1_ref, x_ref, w1_ref, b1_ref, w2_ref, b2_ref,
             lnw_ref, lnb_ref, gnw_ref, gnb_ref, gms_ref,
             out_ref, h_ref, acc_ref):
        k = pl.program_id(0)

        @pl.when(k < nb)
        def _mlp():
            agg = p0_ref[...] + p1_ref[...]
            h1 = jnp.dot(agg, w1_ref[...], preferred_element_type=jnp.float32)
            h1 = jnp.maximum(h1 + b1_ref[...], 0.0)
            h = jnp.dot(h1, w2_ref[...], preferred_element_type=jnp.float32)
            h = h + b2_ref[...]
            h_ref[pl.ds(k * br, br), :] = h

            @pl.when(k == 0)
            def _init():
                acc_ref[...] = jnp.zeros_like(acc_ref)

            acc_ref[0:1, :] += jnp.sum(h, axis=0, keepdims=True)
            acc_ref[1:2, :] += jnp.sum(h * h, axis=0, keepdims=True)

        @pl.when(k >= nb)
        def _norm():
            s1 = acc_ref[0:1, :]
            s2 = acc_ref[1:2, :]
            mean = jnp.sum(s1) * inv_nd
            var = jnp.sum(s2) * inv_nd - mean * mean
            a = lnw_ref[...] * lax.rsqrt(var + 1e-5)
            b = lnb_ref[...] - mean * a
            m = a * (s1 * inv_n) + b
            beta = b - m * gms_ref[...]
            v = (a * a * (s2 * inv_n) + 2.0 * a * beta * (s1 * inv_n)
                 + beta * beta)
            scale = gnw_ref[...] * lax.rsqrt(v + 1e-5)
            A = a * scale
            B = beta * scale + gnb_ref[...]
            h = h_ref[pl.ds((k - nb) * br, br), :]
            out_ref[...] = jax.nn.gelu(h * A + B) + x_ref[...]

    return pl.pallas_call(
        body,
        grid=(2 * nb,),
        in_specs=[
            pl.BlockSpec((br, d), lambda k: (jnp.minimum(k, nb - 1), 0)),
            pl.BlockSpec((br, d), lambda k: (jnp.minimum(k, nb - 1) + nb, 0)),
            pl.BlockSpec((br, d), lambda k: (jnp.maximum(k - nb, 0), 0)),
            pl.BlockSpec((d, 2 * d), lambda k: (0, 0)),
            pl.BlockSpec((1, 2 * d), lambda k: (0, 0)),
            pl.BlockSpec((2 * d, d), lambda k: (0, 0)),
            pl.BlockSpec((1, d), lambda k: (0, 0)),
            pl.BlockSpec((1, d), lambda k: (0, 0)),
            pl.BlockSpec((1, d), lambda k: (0, 0)),
            pl.BlockSpec((1, d), lambda k: (0, 0)),
            pl.BlockSpec((1, d), lambda k: (0, 0)),
            pl.BlockSpec((1, d), lambda k: (0, 0)),
        ],
        out_specs=pl.BlockSpec((br, d), lambda k: (jnp.maximum(k - nb, 0), 0)),
        out_shape=jax.ShapeDtypeStruct((n, d), jnp.float32),
        scratch_shapes=[
            pltpu.VMEM((n, d), jnp.float32),
            pltpu.VMEM((8, d), jnp.float32),
        ],
    )(parts, parts, x, W1, b1, W2, b2, ln_w, ln_b, gn_w, gn_b, gn_ms)


def kernel(x, edge_index, edge_attr, W1, b1, W2, b2,
           ln_weight, ln_bias, gn_weight, gn_bias, gn_mean_scale):
    n, d = x.shape
    e = edge_attr.shape[0]
    idx4d = jnp.transpose(
        edge_index.reshape(2, _NW, e // (_NW * _CH), _CH), (1, 2, 0, 3))

    parts = _sc_aggregate(x, idx4d, edge_attr)

    br = 1000
    return _mlp_norm_gelu_residual(
        parts, x, W1, b1.reshape(1, -1), W2, b2.reshape(1, -1),
        ln_weight.reshape(1, -1), ln_bias.reshape(1, -1),
        gn_weight.reshape(1, -1), gn_bias.reshape(1, -1),
        gn_mean_scale.reshape(1, -1), n, d, br)


# speedup vs baseline: 2.2491x; 2.2491x over previous
import functools

import jax
import jax.numpy as jnp
from jax import lax
from jax.experimental import pallas as pl
from jax.experimental.pallas import tpu as pltpu
from jax.experimental.pallas import tpu_sc as plsc

_NC, _NS = 2, 16
_NW = _NC * _NS
_CH = 80
_LANES = 16
_ZB = 8
_R = 4


def _sc_aggregate(x, idx4d, edge_attr):
    n, d = x.shape
    e = edge_attr.shape[0]
    ept = e // _NW
    nchunk = ept // _CH
    rpt = -(-(n // _NS) // _ZB) * _ZB
    owner = n // rpt
    rem = n % rpt
    ngrp = d // _LANES

    mesh = plsc.VectorSubcoreMesh(core_axis_name="c", subcore_axis_name="s")

    @functools.partial(
        pl.kernel,
        out_type=jax.ShapeDtypeStruct((_NC * n, d), jnp.float32),
        mesh=mesh,
        scratch_types=[
            pltpu.VMEM((_R, 2, _CH), jnp.int32),
            pltpu.VMEM((3 * _CH, d), jnp.float32),
            pltpu.VMEM_SHARED((_NS * rpt, d), jnp.float32),
            pltpu.SemaphoreType.DMA,
            pltpu.SemaphoreType.DMA,
            pltpu.SemaphoreType.DMA,
            pltpu.SemaphoreType.DMA,
        ],
    )
    def agg_kernel(x_hbm, idx_hbm, ea_hbm, out_hbm,
                   idx_v, rows_v, acc_sh,
                   sem_idx, sem_in, sem_scr, sem_sce):
        c = lax.axis_index("c")
        s = lax.axis_index("s")
        wid = c * _NS + s

        def slab(t):
            return rows_v.at[pl.ds(t * _CH, _CH), :]

        def zrow(r, carry):
            for k in range(ngrp):
                rows_v[r, pl.ds(k * _LANES, _LANES)] = jnp.zeros(
                    (_LANES,), jnp.float32)
            return carry
        lax.fori_loop(0, _CH, zrow, 0)
        row0 = s * rpt
        nfull = rpt // _CH
        for t in range(nfull):
            pltpu.sync_copy(slab(0), acc_sh.at[pl.ds(row0 + t * _CH, _CH), :])
        if rpt % _CH:
            pltpu.sync_copy(
                rows_v.at[pl.ds(0, rpt % _CH), :],
                acc_sh.at[pl.ds(row0 + nfull * _CH, rpt % _CH), :])
        plsc.subcore_barrier()

        base = wid * ept

        def ea_copy(k, t):
            return pltpu.make_async_copy(
                ea_hbm.at[pl.ds(base + k * _CH, _CH), :], slab(t), sem_in)

        def gather_copy(islot, t):
            return pltpu.make_async_copy(
                x_hbm.at[idx_v.at[islot, 0]], slab(t), sem_in)

        def pair_wait():
            pltpu.make_async_copy(
                ea_hbm.at[pl.ds(0, 2 * _CH), :],
                rows_v.at[pl.ds(0, 2 * _CH), :], sem_in).wait()

        def sc_drain(sem):
            pltpu.make_async_copy(
                ea_hbm.at[pl.ds(0, _CH), :], slab(0), sem).wait()

        def idx_copy(k, slot):
            return pltpu.make_async_copy(idx_hbm.at[wid, k], idx_v.at[slot],
                                         sem_idx)

        pltpu.sync_copy(idx_hbm.at[wid, 0], idx_v.at[0])
        gather_copy(0, 0).start()
        ea_copy(0, 2).start()
        idx_copy(1, 1).start()
        idx_copy(2, 2).start()

        def chunk(j, carry):
            g = j % 3
            e2 = (j + 2) % 3
            g1 = (j + 1) % 3
            i = j % _R
            i1 = (j + 1) % _R
            more = j + 1 < nchunk

            @pl.when(j >= 1)
            def _drain_scr():
                sc_drain(sem_scr)
                ea_copy(j, e2).start()

            @pl.when(more)
            def _wait_idx():
                idx_copy(j + 1, i1).wait()

            @pl.when(j + 3 < nchunk)
            def _prefetch_idx():
                idx_copy(j + 3, (j + 3) % _R).start()

            pair_wait()

            @pl.when(j >= 1)
            def _drain_sce():
                sc_drain(sem_sce)

            @pl.when(more)
            def _next_gather():
                gather_copy(i1, g1).start()

            pltpu.async_copy(slab(g), acc_sh.at[idx_v.at[i, 1]], sem_scr,
                             add=True)
            pltpu.async_copy(slab(e2), acc_sh.at[idx_v.at[i, 1]], sem_sce,
                             add=True)
            return carry
        lax.fori_loop(0, nchunk, chunk, 0)

        sc_drain(sem_scr)
        sc_drain(sem_sce)

        plsc.subcore_barrier()

        @pl.when(s < owner)
        def _full():
            pltpu.sync_copy(acc_sh.at[pl.ds(row0, rpt), :],
                            out_hbm.at[pl.ds(c * n + row0, rpt), :])

        if rem:
            @pl.when(s == owner)
            def _ragged():
                pltpu.sync_copy(acc_sh.at[pl.ds(row0, rem), :],
                                out_hbm.at[pl.ds(c * n + row0, rem), :])

    return agg_kernel(x, idx4d, edge_attr)


def _mlp_norm_gelu_residual(parts, x, W1, b1, W2, b2,
                            ln_w, ln_b, gn_w, gn_b, gn_ms, n, d, br):
    nb = n // br
    inv_nd = 1.0 / (n * d)
    inv_n = 1.0 / n

    def body(p0_ref, p1_ref, x_ref, w1_ref, b1_ref, w2_ref, b2_ref,
             lnw_ref, lnb_ref, gnw_ref, gnb_ref, gms_ref,
             out_ref, h_ref, acc_ref):
        k = pl.program_id(0)

        @pl.when(k < nb)
        def _mlp():
            agg = p0_ref[...] + p1_ref[...]
            h1 = jnp.dot(agg, w1_ref[...], preferred_element_type=jnp.float32)
            h1 = jnp.maximum(h1 + b1_ref[...], 0.0)
            h = jnp.dot(h1, w2_ref[...], preferred_element_type=jnp.float32)
            h = h + b2_ref[...]
            h_ref[pl.ds(k * br, br), :] = h

            @pl.when(k == 0)
            def _init():
                acc_ref[...] = jnp.zeros_like(acc_ref)

            acc_ref[0:1, :] += jnp.sum(h, axis=0, keepdims=True)
            acc_ref[1:2, :] += jnp.sum(h * h, axis=0, keepdims=True)

        @pl.when(k >= nb)
        def _norm():
            s1 = acc_ref[0:1, :]
            s2 = acc_ref[1:2, :]
            mean = jnp.sum(s1) * inv_nd
            var = jnp.sum(s2) * inv_nd - mean * mean
            a = lnw_ref[...] * lax.rsqrt(var + 1e-5)
            b = lnb_ref[...] - mean * a
            m = a * (s1 * inv_n) + b
            beta = b - m * gms_ref[...]
            v = (a * a * (s2 * inv_n) + 2.0 * a * beta * (s1 * inv_n)
                 + beta * beta)
            scale = gnw_ref[...] * lax.rsqrt(v + 1e-5)
            A = a * scale
            B = beta * scale + gnb_ref[...]
            h = h_ref[pl.ds((k - nb) * br, br), :]
            out_ref[...] = jax.nn.gelu(h * A + B) + x_ref[...]

    return pl.pallas_call(
        body,
        grid=(2 * nb,),
        in_specs=[
            pl.BlockSpec((br, d), lambda k: (jnp.minimum(k, nb - 1), 0)),
            pl.BlockSpec((br, d), lambda k: (jnp.minimum(k, nb - 1) + nb, 0)),
            pl.BlockSpec((br, d), lambda k: (jnp.maximum(k - nb, 0), 0)),
            pl.BlockSpec((d, 2 * d), lambda k: (0, 0)),
            pl.BlockSpec((1, 2 * d), lambda k: (0, 0)),
            pl.BlockSpec((2 * d, d), lambda k: (0, 0)),
            pl.BlockSpec((1, d), lambda k: (0, 0)),
            pl.BlockSpec((1, d), lambda k: (0, 0)),
            pl.BlockSpec((1, d), lambda k: (0, 0)),
            pl.BlockSpec((1, d), lambda k: (0, 0)),
            pl.BlockSpec((1, d), lambda k: (0, 0)),
            pl.BlockSpec((1, d), lambda k: (0, 0)),
        ],
        out_specs=pl.BlockSpec((br, d), lambda k: (jnp.maximum(k - nb, 0), 0)),
        out_shape=jax.ShapeDtypeStruct((n, d), jnp.float32),
        scratch_shapes=[
            pltpu.VMEM((n, d), jnp.float32),
            pltpu.VMEM((8, d), jnp.float32),
        ],
    )(parts, parts, x, W1, b1, W2, b2, ln_w, ln_b, gn_w, gn_b, gn_ms)


def kernel(x, edge_index, edge_attr, W1, b1, W2, b2,
           ln_weight, ln_bias, gn_weight, gn_bias, gn_mean_scale):
    n, d = x.shape
    e = edge_attr.shape[0]
    idx4d = jnp.transpose(
        edge_index.reshape(2, _NW, e // (_NW * _CH), _CH), (1, 2, 0, 3))

    parts = _sc_aggregate(x, idx4d, edge_attr)

    br = 1000
    return _mlp_norm_gelu_residual(
        parts, x, W1, b1.reshape(1, -1), W2, b2.reshape(1, -1),
        ln_weight.reshape(1, -1), ln_bias.reshape(1, -1),
        gn_weight.reshape(1, -1), gn_bias.reshape(1, -1),
        gn_mean_scale.reshape(1, -1), n, d, br)


# speedup vs baseline: 2.6188x; 1.1644x over previous
import functools

import jax
import jax.numpy as jnp
from jax import lax
from jax.experimental import pallas as pl
from jax.experimental.pallas import tpu as pltpu
from jax.experimental.pallas import tpu_sc as plsc

_NC, _NS = 2, 16
_NW = _NC * _NS
_CH = 80
_LANES = 16
_ZB = 8
_R = 4


def _sc_aggregate(x, idx4d, edge_attr):
    n, d = x.shape
    e = edge_attr.shape[0]
    ept = e // _NW
    nchunk = ept // _CH
    rpt = -(-(n // _NS) // _ZB) * _ZB
    owner = n // rpt
    rem = n % rpt
    ngrp = d // _LANES

    mesh = plsc.VectorSubcoreMesh(core_axis_name="c", subcore_axis_name="s")

    @functools.partial(
        pl.kernel,
        out_type=jax.ShapeDtypeStruct((_NC * n, d), jnp.float32),
        mesh=mesh,
        scratch_types=[
            pltpu.VMEM((_R, 2, _CH), jnp.int32),
            pltpu.VMEM((3 * _CH, d), jnp.float32),
            pltpu.VMEM_SHARED((_NS * rpt, d), jnp.float32),
            pltpu.SemaphoreType.DMA,
            pltpu.SemaphoreType.DMA,
            pltpu.SemaphoreType.DMA,
            pltpu.SemaphoreType.DMA,
        ],
    )
    def agg_kernel(x_hbm, idx_hbm, ea_hbm, out_hbm,
                   idx_v, rows_v, acc_sh,
                   sem_idx, sem_in, sem_scr, sem_sce):
        c = lax.axis_index("c")
        s = lax.axis_index("s")
        wid = c * _NS + s

        def slab(t):
            return rows_v.at[pl.ds(t * _CH, _CH), :]

        def zrow(r, carry):
            for k in range(ngrp):
                rows_v[r, pl.ds(k * _LANES, _LANES)] = jnp.zeros(
                    (_LANES,), jnp.float32)
            return carry
        lax.fori_loop(0, _CH, zrow, 0)
        row0 = s * rpt
        nfull = rpt // _CH
        for t in range(nfull):
            pltpu.sync_copy(slab(0), acc_sh.at[pl.ds(row0 + t * _CH, _CH), :])
        if rpt % _CH:
            pltpu.sync_copy(
                rows_v.at[pl.ds(0, rpt % _CH), :],
                acc_sh.at[pl.ds(row0 + nfull * _CH, rpt % _CH), :])
        plsc.subcore_barrier()

        base = wid * ept

        def ea_copy(k, t):
            return pltpu.make_async_copy(
                ea_hbm.at[pl.ds(base + k * _CH, _CH), :], slab(t), sem_in)

        def gather_copy(islot, t):
            return pltpu.make_async_copy(
                x_hbm.at[idx_v.at[islot, 0]], slab(t), sem_in)

        def pair_wait():
            pltpu.make_async_copy(
                ea_hbm.at[pl.ds(0, 2 * _CH), :],
                rows_v.at[pl.ds(0, 2 * _CH), :], sem_in).wait()

        def sc_drain(sem):
            pltpu.make_async_copy(
                ea_hbm.at[pl.ds(0, _CH), :], slab(0), sem).wait()

        def idx_copy(k, slot):
            return pltpu.make_async_copy(idx_hbm.at[wid, k], idx_v.at[slot],
                                         sem_idx)

        pltpu.sync_copy(idx_hbm.at[wid, 0], idx_v.at[0])
        gather_copy(0, 0).start()
        ea_copy(0, 2).start()
        idx_copy(1, 1).start()
        idx_copy(2, 2).start()

        def chunk(j, carry):
            g = j % 3
            e2 = (j + 2) % 3
            g1 = (j + 1) % 3
            i = j % _R
            i1 = (j + 1) % _R
            more = j + 1 < nchunk

            @pl.when(j >= 1)
            def _drain_scr():
                sc_drain(sem_scr)
                ea_copy(j, e2).start()

            @pl.when(more)
            def _wait_idx():
                idx_copy(j + 1, i1).wait()

            @pl.when(j + 3 < nchunk)
            def _prefetch_idx():
                idx_copy(j + 3, (j + 3) % _R).start()

            pair_wait()

            @pl.when(more)
            def _next_gather():
                gather_copy(i1, g1).start()

            @functools.partial(plsc.parallel_loop, 0, _CH, unroll=8)
            def _add(r):
                for k in range(ngrp):
                    sl = pl.ds(k * _LANES, _LANES)
                    rows_v[g * _CH + r, sl] = (rows_v[g * _CH + r, sl]
                                               + rows_v[e2 * _CH + r, sl])

            pltpu.async_copy(slab(g), acc_sh.at[idx_v.at[i, 1]], sem_scr,
                             add=True)
            return carry
        lax.fori_loop(0, nchunk, chunk, 0)

        sc_drain(sem_scr)

        plsc.subcore_barrier()

        @pl.when(s < owner)
        def _full():
            pltpu.sync_copy(acc_sh.at[pl.ds(row0, rpt), :],
                            out_hbm.at[pl.ds(c * n + row0, rpt), :])

        if rem:
            @pl.when(s == owner)
            def _ragged():
                pltpu.sync_copy(acc_sh.at[pl.ds(row0, rem), :],
                                out_hbm.at[pl.ds(c * n + row0, rem), :])

    return agg_kernel(x, idx4d, edge_attr)


def _mlp_norm_gelu_residual(parts, x, W1, b1, W2, b2,
                            ln_w, ln_b, gn_w, gn_b, gn_ms, n, d, br):
    nb = n // br
    inv_nd = 1.0 / (n * d)
    inv_n = 1.0 / n

    def body(p0_ref, p1_ref, x_ref, w1_ref, b1_ref, w2_ref, b2_ref,
             lnw_ref, lnb_ref, gnw_ref, gnb_ref, gms_ref,
             out_ref, h_ref, acc_ref):
        k = pl.program_id(0)

        @pl.when(k < nb)
        def _mlp():
            agg = p0_ref[...] + p1_ref[...]
            h1 = jnp.dot(agg, w1_ref[...], preferred_element_type=jnp.float32)
            h1 = jnp.maximum(h1 + b1_ref[...], 0.0)
            h = jnp.dot(h1, w2_ref[...], preferred_element_type=jnp.float32)
            h = h + b2_ref[...]
            h_ref[pl.ds(k * br, br), :] = h

            @pl.when(k == 0)
            def _init():
                acc_ref[...] = jnp.zeros_like(acc_ref)

            acc_ref[0:1, :] += jnp.sum(h, axis=0, keepdims=True)
            acc_ref[1:2, :] += jnp.sum(h * h, axis=0, keepdims=True)

        @pl.when(k >= nb)
        def _norm():
            s1 = acc_ref[0:1, :]
            s2 = acc_ref[1:2, :]
            mean = jnp.sum(s1) * inv_nd
            var = jnp.sum(s2) * inv_nd - mean * mean
            a = lnw_ref[...] * lax.rsqrt(var + 1e-5)
            b = lnb_ref[...] - mean * a
            m = a * (s1 * inv_n) + b
            beta = b - m * gms_ref[...]
            v = (a * a * (s2 * inv_n) + 2.0 * a * beta * (s1 * inv_n)
                 + beta * beta)
            scale = gnw_ref[...] * lax.rsqrt(v + 1e-5)
            A = a * scale
            B = beta * scale + gnb_ref[...]
            h = h_ref[pl.ds((k - nb) * br, br), :]
            out_ref[...] = jax.nn.gelu(h * A + B) + x_ref[...]

    return pl.pallas_call(
        body,
        grid=(2 * nb,),
        in_specs=[
            pl.BlockSpec((br, d), lambda k: (jnp.minimum(k, nb - 1), 0)),
            pl.BlockSpec((br, d), lambda k: (jnp.minimum(k, nb - 1) + nb, 0)),
            pl.BlockSpec((br, d), lambda k: (jnp.maximum(k - nb, 0), 0)),
            pl.BlockSpec((d, 2 * d), lambda k: (0, 0)),
            pl.BlockSpec((1, 2 * d), lambda k: (0, 0)),
            pl.BlockSpec((2 * d, d), lambda k: (0, 0)),
            pl.BlockSpec((1, d), lambda k: (0, 0)),
            pl.BlockSpec((1, d), lambda k: (0, 0)),
            pl.BlockSpec((1, d), lambda k: (0, 0)),
            pl.BlockSpec((1, d), lambda k: (0, 0)),
            pl.BlockSpec((1, d), lambda k: (0, 0)),
            pl.BlockSpec((1, d), lambda k: (0, 0)),
        ],
        out_specs=pl.BlockSpec((br, d), lambda k: (jnp.maximum(k - nb, 0), 0)),
        out_shape=jax.ShapeDtypeStruct((n, d), jnp.float32),
        scratch_shapes=[
            pltpu.VMEM((n, d), jnp.float32),
            pltpu.VMEM((8, d), jnp.float32),
        ],
    )(parts, parts, x, W1, b1, W2, b2, ln_w, ln_b, gn_w, gn_b, gn_ms)


def kernel(x, edge_index, edge_attr, W1, b1, W2, b2,
           ln_weight, ln_bias, gn_weight, gn_bias, gn_mean_scale):
    n, d = x.shape
    e = edge_attr.shape[0]
    idx4d = jnp.transpose(
        edge_index.reshape(2, _NW, e // (_NW * _CH), _CH), (1, 2, 0, 3))

    parts = _sc_aggregate(x, idx4d, edge_attr)

    br = 1000
    return _mlp_norm_gelu_residual(
        parts, x, W1, b1.reshape(1, -1), W2, b2.reshape(1, -1),
        ln_weight.reshape(1, -1), ln_bias.reshape(1, -1),
        gn_weight.reshape(1, -1), gn_bias.reshape(1, -1),
        gn_mean_scale.reshape(1, -1), n, d, br)
